# baseline (speedup 1.0000x reference)
import jax
import jax.numpy as jnp
from jax.experimental import pallas as pl
from jax.experimental.pallas import tpu as pltpu


def _taps():
    taps = []
    for kh in range(7):
        p = (kh + 1) % 2
        rh = (kh - 3 - p) // 2 + 2
        for kw in range(7):
            q = (kw + 1) % 2
            rw = (kw - 3 - q) // 2 + 2
            taps.append((p, q, rh, rw))
    return taps


_TAPS = _taps()


def _gconv_body(xph_ref, wg_ref, fout_ref, stats_ref):
    wg = wg_ref[...]

    def chunk(i, acc):
        slabs = {}
        for p in range(2):
            for q in range(2):
                s16 = xph_ref[p, q, :, pl.ds(i, 2), :, :]
                slabs[(p, q)] = s16.reshape(15, 16, 196)
        parts = []
        for (p, q, rh, rw) in _TAPS:
            sl = jax.lax.slice(slabs[(p, q)], (0, rh, rw),
                               (15, rh + 8, rw + 192))
            parts.append(sl)
        xall = jnp.concatenate(parts, axis=0)
        xall = xall.reshape(49 * 15, 8 * 192)
        y = jnp.dot(wg, xall, preferred_element_type=jnp.float32)
        fout_ref[:, i, :, :] = y.reshape(64, 8, 192)
        s = jnp.sum(y, axis=1, keepdims=True)
        sq = jnp.sum(y * y, axis=1, keepdims=True)
        return acc + jnp.concatenate([s, sq], axis=1)

    stats_ref[...] = jax.lax.fori_loop(
        0, 24, chunk, jnp.zeros((64, 2), jnp.float32))


def _pool_body(fout_ref, stats_ref, gam_ref, bet_ref, feat_ref):
    n = float(8 * 192 * 192)
    st = jnp.sum(stats_ref[...], axis=0)
    mu = st[:, 0:1] * (1.0 / n)
    var = st[:, 1:2] * (1.0 / n) - mu * mu
    scale = gam_ref[...] * jax.lax.rsqrt(var + 1e-5)
    shift = bet_ref[...] - mu * scale
    sc3 = scale.reshape(64, 1, 1)
    sh3 = shift.reshape(64, 1, 1)

    def grp(g, acc):
        cur = fout_ref[:, pl.ds(g, 1), :, :].reshape(64, 8, 192)
        pg = fout_ref[:, pl.ds(jnp.maximum(g - 1, 0), 1), :, :]
        prev = jax.lax.slice(pg.reshape(64, 8, 192), (0, 7, 0), (64, 8, 192))
        prev = jnp.where(g == 0, 0.0, prev)
        y = jnp.concatenate([prev, cur], axis=1)
        y = jnp.maximum(y * sc3 + sh3, 0.0)
        ypc = jnp.pad(y, ((0, 0), (0, 0), (1, 1)))
        cm = jnp.maximum(
            jnp.maximum(jax.lax.slice(ypc, (0, 0, 0), (64, 9, 192)),
                        jax.lax.slice(ypc, (0, 0, 1), (64, 9, 193))),
            jax.lax.slice(ypc, (0, 0, 2), (64, 9, 194)))
        s = acc
        for j in range(4):
            r0 = jax.lax.slice(cm, (0, 2 * j, 0), (64, 2 * j + 1, 192))
            r1 = jax.lax.slice(cm, (0, 2 * j + 1, 0), (64, 2 * j + 2, 192))
            r2 = jax.lax.slice(cm, (0, 2 * j + 2, 0), (64, 2 * j + 3, 192))
            pj = jnp.maximum(jnp.maximum(r0, r1), r2).reshape(64, 192)
            s = s + pj
        return s

    racc = jax.lax.fori_loop(0, 24, grp, jnp.zeros((64, 192), jnp.float32))
    even = (jax.lax.broadcasted_iota(jnp.int32, (192, 1), 0) % 2
            == 0).astype(jnp.float32)
    feat_ref[...] = jnp.dot(racc, even,
                            preferred_element_type=jnp.float32) * (
                                1.0 / (96 * 96))


def _route_body(feat_ref, fcw_ref, fcb_ref, oh_ref, aux_ref):
    bsz = feat_ref.shape[0]
    feat = feat_ref[...].reshape(bsz, 64)
    logits = jax.lax.dot_general(
        feat, fcw_ref[...], (((1,), (1,)), ((), ())),
        preferred_element_type=jnp.float32) + fcb_ref[...]
    l0 = logits[:, 0:1]
    l1 = logits[:, 1:2]
    l2 = logits[:, 2:3]
    o0 = jnp.logical_and(l0 >= l1, l0 >= l2)
    o1 = jnp.logical_and(l1 > l0, l1 >= l2)
    o2 = jnp.logical_and(l2 > l0, l2 > l1)
    onehot = jnp.concatenate([o0.astype(jnp.float32),
                              o1.astype(jnp.float32),
                              o2.astype(jnp.float32)], axis=1)
    mx = jnp.max(logits, axis=1, keepdims=True)
    e = jnp.exp(logits - mx)
    p = e / jnp.sum(e, axis=1, keepdims=True)
    density = jnp.sum(onehot, axis=0, keepdims=True) * (1.0 / bsz)
    dproxy = jnp.sum(p, axis=0, keepdims=True) * (1.0 / bsz)
    aux_ref[...] = jnp.sum(density * dproxy, axis=1,
                           keepdims=True) * (3.0 * 0.01)
    oh_ref[...] = onehot.reshape(bsz, 1, 3)


def _exp_body(xp_ref, oh_ref, w0_ref, w1_ref, w2_ref, bex_ref, out_ref):
    g = oh_ref[...].reshape(1, 3)
    g0 = jax.lax.slice(g, (0, 0), (1, 1))
    g1 = jax.lax.slice(g, (0, 1), (1, 2))
    g2 = jax.lax.slice(g, (0, 2), (1, 3))
    wc = g0 * w0_ref[...] + g1 * w1_ref[...] + g2 * w2_ref[...]
    bc = jnp.dot(g, bex_ref[...], preferred_element_type=jnp.float32)
    out_ref[...] = jnp.dot(xp_ref[...], wc,
                           preferred_element_type=jnp.float32) + bc


def kernel(x, conv_w, bn_gamma, bn_beta, fc_w, fc_b, exp_w, exp_b):
    B, C, T, H, W = x.shape
    CT = C * T
    x2 = x.reshape(B, CT, H, W)

    xpad = jnp.pad(x2, ((0, 0), (0, 0), (4, 12), (4, 4)))
    xph = xpad.reshape(B, CT, 200, 2, 196, 2).transpose(
        0, 3, 5, 1, 2, 4).reshape(B, 2, 2, CT, 25, 8, 196)
    wg = conv_w.transpose(0, 2, 3, 1).reshape(64, 49 * CT)

    fout, stats = pl.pallas_call(
        _gconv_body,
        grid=(B,),
        in_specs=[
            pl.BlockSpec((None, 2, 2, CT, 25, 8, 196),
                         lambda b: (b, 0, 0, 0, 0, 0, 0)),
            pl.BlockSpec((64, 49 * CT), lambda b: (0, 0)),
        ],
        out_specs=[
            pl.BlockSpec((None, 64, 24, 8, 192), lambda b: (b, 0, 0, 0, 0)),
            pl.BlockSpec((None, 64, 2), lambda b: (b, 0, 0)),
        ],
        out_shape=[
            jax.ShapeDtypeStruct((B, 64, 24, 8, 192), jnp.float32),
            jax.ShapeDtypeStruct((B, 64, 2), jnp.float32),
        ],
        compiler_params=pltpu.CompilerParams(
            dimension_semantics=("parallel",)),
    )(xph, wg)

    feat = pl.pallas_call(
        _pool_body,
        grid=(B,),
        in_specs=[
            pl.BlockSpec((None, 64, 24, 8, 192), lambda b: (b, 0, 0, 0, 0)),
            pl.BlockSpec((B, 64, 2), lambda b: (0, 0, 0)),
            pl.BlockSpec((64, 1), lambda b: (0, 0)),
            pl.BlockSpec((64, 1), lambda b: (0, 0)),
        ],
        out_specs=pl.BlockSpec((None, 64, 1), lambda b: (b, 0, 0)),
        out_shape=jax.ShapeDtypeStruct((B, 64, 1), jnp.float32),
        compiler_params=pltpu.CompilerParams(
            dimension_semantics=("parallel",)),
    )(fout, stats, bn_gamma.reshape(64, 1), bn_beta.reshape(64, 1))

    onehot, aux = pl.pallas_call(
        _route_body,
        out_shape=[
            jax.ShapeDtypeStruct((B, 1, 3), jnp.float32),
            jax.ShapeDtypeStruct((1, 1), jnp.float32),
        ],
    )(feat, fc_w, fc_b.reshape(1, 3))

    x2p = x2.reshape(B, CT, 96, 4, 96, 4).transpose(
        0, 2, 4, 1, 3, 5).reshape(B, 96 * 96, 16 * CT)
    wexs = exp_w.reshape(3, 5, 16 * CT).transpose(0, 2, 1)

    headsf = pl.pallas_call(
        _exp_body,
        grid=(B,),
        in_specs=[
            pl.BlockSpec((None, 96 * 96, 16 * CT), lambda b: (b, 0, 0)),
            pl.BlockSpec((None, 1, 3), lambda b: (b, 0, 0)),
            pl.BlockSpec((16 * CT, 5), lambda b: (0, 0)),
            pl.BlockSpec((16 * CT, 5), lambda b: (0, 0)),
            pl.BlockSpec((16 * CT, 5), lambda b: (0, 0)),
            pl.BlockSpec((3, 5), lambda b: (0, 0)),
        ],
        out_specs=pl.BlockSpec((None, 96 * 96, 5), lambda b: (b, 0, 0)),
        out_shape=jax.ShapeDtypeStruct((B, 96 * 96, 5), jnp.float32),
        compiler_params=pltpu.CompilerParams(
            dimension_semantics=("parallel",)),
    )(x2p, onehot, wexs[0], wexs[1], wexs[2], exp_b)

    heads = headsf.reshape(B, 96, 96, 5).transpose(0, 3, 1, 2)
    return heads, aux.reshape(())
